# baseline (speedup 1.0000x reference)
import functools

import jax
import jax.numpy as jnp
from jax import lax
from jax.experimental import pallas as pl
from jax.experimental.pallas import tpu as pltpu
from jax.experimental.pallas import tpu_sc as plsc

N = 10000
E = 320000
D = 128
DH = D // 2

NC = 2
NS = 16
EPT = E // NS
CH = 400
NCHUNK = EPT // CH
SUB = 80
NSUB = CH // SUB
NRB = 3
NIB = 4
RPT = 624
TAIL = N - NS * RPT
ZR = 104
LANES = 16

_mesh = plsc.VectorSubcoreMesh(core_axis_name="c", subcore_axis_name="s")


@functools.partial(
    pl.kernel,
    out_type=jax.ShapeDtypeStruct((NC * N, DH), jnp.float32),
    mesh=_mesh,
    scratch_types=[
        pltpu.VMEM((NIB, NSUB, SUB), jnp.int32),
        pltpu.VMEM((NIB, NSUB, SUB), jnp.int32),
        pltpu.VMEM((NIB, CH), jnp.float32),
        pltpu.VMEM((NRB, CH, DH), jnp.float32),
        pltpu.VMEM((ZR, DH), jnp.float32),
        pltpu.VMEM_SHARED((N, DH), jnp.float32),
        pltpu.SemaphoreType.DMA,
        pltpu.SemaphoreType.DMA,
        pltpu.SemaphoreType.DMA,
    ],
    compiler_params=pltpu.CompilerParams(use_tc_tiling_on_sc=False),
)
def _sc_agg(xs_hbm, src_hbm, dst_hbm, wt_hbm, out_hbm,
            src_v, dst_v, wt_v, rows_v, zero_v, agg_sh, isem, gsem, ssem):
    c = lax.axis_index("c")
    s = lax.axis_index("s")
    ebase = s * EPT

    zvec = jnp.zeros((LANES,), jnp.float32)

    def zrow(r, carry):
        for j in range(DH // LANES):
            zero_v[r, pl.ds(j * LANES, LANES)] = zvec
        return carry

    lax.fori_loop(0, ZR, zrow, 0)
    row0 = s * RPT
    for t in range(RPT // ZR):
        pltpu.async_copy(zero_v, agg_sh.at[pl.ds(row0 + t * ZR, ZR)], ssem)

    @pl.when(s == NS - 1)
    def _zero_tail():
        pltpu.sync_copy(zero_v.at[pl.ds(0, TAIL)],
                        agg_sh.at[pl.ds(NS * RPT, TAIL)])

    def _load_idx(k, ib):
        base = ebase + k * CH
        for t in range(NSUB):
            pltpu.async_copy(src_hbm.at[pl.ds(base + t * SUB, SUB)],
                             src_v.at[ib, t], isem)
            pltpu.async_copy(dst_hbm.at[pl.ds(base + t * SUB, SUB)],
                             dst_v.at[ib, t], isem)
        pltpu.async_copy(wt_hbm.at[pl.ds(base, CH)], wt_v.at[ib], isem)

    def _wait_idx(ib):
        for t in range(NSUB):
            pltpu.make_async_copy(src_hbm.at[pl.ds(0, SUB)],
                                  src_v.at[ib, t], isem).wait()
            pltpu.make_async_copy(dst_hbm.at[pl.ds(0, SUB)],
                                  dst_v.at[ib, t], isem).wait()
        pltpu.make_async_copy(wt_hbm.at[pl.ds(0, CH)], wt_v.at[ib],
                              isem).wait()

    def _rebase(ib):
        cN = c * N
        for t in range(NSUB):
            for u in range(SUB // LANES):
                sl = pl.ds(u * LANES, LANES)
                src_v[ib, t, sl] = src_v[ib, t, sl] + cN

    def _fire_gathers(ib, rb):
        for t in range(NSUB):
            pltpu.async_copy(xs_hbm.at[src_v.at[ib, t]],
                             rows_v.at[rb, pl.ds(t * SUB, SUB)], gsem)

    def _drain_scatters():
        pltpu.make_async_copy(xs_hbm.at[pl.ds(0, CH)], rows_v.at[0],
                              ssem).wait()

    _load_idx(0, 0)
    _load_idx(1, 1)
    _load_idx(2, 2)
    _wait_idx(0)
    _rebase(0)
    _fire_gathers(0, 0)
    _wait_idx(1)
    _rebase(1)
    _fire_gathers(1, 1)

    for t in range(RPT // ZR):
        pltpu.make_async_copy(xs_hbm.at[pl.ds(0, ZR)], zero_v, ssem).wait()
    plsc.subcore_barrier()

    def chunk(k, carry):
        rb = lax.rem(k, NRB)
        ib = lax.rem(k, NIB)
        ib2 = lax.rem(k + 2, NIB)

        def egroup(g, ecarry):
            e0 = g * LANES
            wv = wt_v[ib, pl.ds(e0, LANES)]
            for l in range(LANES):
                w = wv[l]
                for j in range(DH // LANES):
                    rows_v[rb, e0 + l, pl.ds(j * LANES, LANES)] = (
                        rows_v[rb, e0 + l, pl.ds(j * LANES, LANES)] * w)
            return ecarry

        for t in range(NSUB):
            pltpu.make_async_copy(xs_hbm.at[pl.ds(0, SUB)],
                                  rows_v.at[rb, pl.ds(t * SUB, SUB)],
                                  gsem).wait()
            lax.fori_loop(t * (SUB // LANES), (t + 1) * (SUB // LANES),
                          egroup, 0)
            pltpu.async_copy(rows_v.at[rb, pl.ds(t * SUB, SUB)],
                             agg_sh.at[dst_v.at[ib, t]], ssem, add=True)

        @pl.when(k + 2 < NCHUNK)
        def _prep_next():
            _wait_idx(ib2)
            _rebase(ib2)

            @pl.when(k >= 1)
            def _drain_prev_scatters():
                _drain_scatters()

            _fire_gathers(ib2, lax.rem(k + 2, NRB))

            @pl.when(k + 3 < NCHUNK)
            def _load_next_idx():
                _load_idx(k + 3, lax.rem(k + 3, NIB))

        return carry

    lax.fori_loop(0, NCHUNK, chunk, 0)

    _drain_scatters()
    _drain_scatters()
    _drain_scatters()
    plsc.subcore_barrier()

    pltpu.sync_copy(agg_sh.at[pl.ds(row0, RPT)],
                    out_hbm.at[pl.ds(c * N + row0, RPT)])

    @pl.when(s == NS - 1)
    def _write_tail():
        pltpu.sync_copy(agg_sh.at[pl.ds(NS * RPT, TAIL)],
                        out_hbm.at[pl.ds(c * N + NS * RPT, TAIL)])


BM = 1000


def _tc_body(p0_ref, p1_ref, w0_ref, w1_ref, b_ref, o_ref):
    o_ref[...] = (
        jnp.dot(p0_ref[...], w0_ref[...], preferred_element_type=jnp.float32)
        + jnp.dot(p1_ref[...], w1_ref[...], preferred_element_type=jnp.float32)
        + b_ref[...])


def _tc_matmul(partial, W, b):
    nb = N // BM
    return pl.pallas_call(
        _tc_body,
        grid=(nb,),
        in_specs=[
            pl.BlockSpec((BM, DH), lambda i: (i, 0)),
            pl.BlockSpec((BM, DH), lambda i: (i + nb, 0)),
            pl.BlockSpec((DH, D), lambda i: (0, 0)),
            pl.BlockSpec((DH, D), lambda i: (0, 0)),
            pl.BlockSpec((1, D), lambda i: (0, 0)),
        ],
        out_specs=pl.BlockSpec((BM, D), lambda i: (i, 0)),
        out_shape=jax.ShapeDtypeStruct((N, D), jnp.float32),
    )(partial, partial, W[:DH], W[DH:], b.reshape(1, D))


def kernel(x, edge_index, edge_weight, W, b):
    src = edge_index[1].astype(jnp.int32)
    dst = edge_index[0].astype(jnp.int32)
    xs = jnp.concatenate([x[:, :DH], x[:, DH:]], axis=0)
    partial = _sc_agg(xs, src, dst, edge_weight)
    return _tc_matmul(partial, W, b)


# speedup vs baseline: 1.5352x; 1.5352x over previous
import functools

import jax
import jax.numpy as jnp
from jax import lax
from jax.experimental import pallas as pl
from jax.experimental.pallas import tpu as pltpu
from jax.experimental.pallas import tpu_sc as plsc

N = 10000
E = 320000
D = 128
DH = D // 2

NC = 2
NS = 16
EPT = E // NS
CH = 400
NCHUNK = EPT // CH
SUB = 80
NSUB = CH // SUB
NRB = 2
NIB = 3
RPT = 624
TAIL = N - NS * RPT
ZR = 104
LANES = 16

_mesh = plsc.VectorSubcoreMesh(core_axis_name="c", subcore_axis_name="s")


@functools.partial(
    pl.kernel,
    out_type=jax.ShapeDtypeStruct((NC * N, DH), jnp.float32),
    mesh=_mesh,
    scratch_types=[
        pltpu.VMEM((NIB, NSUB, SUB), jnp.int32),
        pltpu.VMEM((NIB, NSUB, SUB), jnp.int32),
        pltpu.VMEM((NIB, CH), jnp.float32),
        pltpu.VMEM((NRB, CH, DH), jnp.float32),
        pltpu.VMEM((ZR, DH), jnp.float32),
        pltpu.VMEM_SHARED((N, DH), jnp.float32),
        pltpu.SemaphoreType.DMA,
        pltpu.SemaphoreType.DMA,
        pltpu.SemaphoreType.DMA,
    ],
    compiler_params=pltpu.CompilerParams(use_tc_tiling_on_sc=False),
)
def _sc_agg(xs_hbm, src_hbm, dst_hbm, wt_hbm, out_hbm,
            src_v, dst_v, wt_v, rows_v, zero_v, agg_sh, isem, gsem, ssem):
    c = lax.axis_index("c")
    s = lax.axis_index("s")
    ebase = s * EPT

    zvec = jnp.zeros((LANES,), jnp.float32)

    def zrow(r, carry):
        for j in range(DH // LANES):
            zero_v[r, pl.ds(j * LANES, LANES)] = zvec
        return carry

    lax.fori_loop(0, ZR, zrow, 0)
    row0 = s * RPT
    for t in range(RPT // ZR):
        pltpu.async_copy(zero_v, agg_sh.at[pl.ds(row0 + t * ZR, ZR)], ssem)

    @pl.when(s == NS - 1)
    def _zero_tail():
        pltpu.sync_copy(zero_v.at[pl.ds(0, TAIL)],
                        agg_sh.at[pl.ds(NS * RPT, TAIL)])

    def _load_idx(k, ib):
        base = ebase + k * CH
        for t in range(NSUB):
            pltpu.async_copy(src_hbm.at[pl.ds(base + t * SUB, SUB)],
                             src_v.at[ib, t], isem)
            pltpu.async_copy(dst_hbm.at[pl.ds(base + t * SUB, SUB)],
                             dst_v.at[ib, t], isem)
        pltpu.async_copy(wt_hbm.at[pl.ds(base, CH)], wt_v.at[ib], isem)

    def _wait_idx(ib):
        for t in range(NSUB):
            pltpu.make_async_copy(src_hbm.at[pl.ds(0, SUB)],
                                  src_v.at[ib, t], isem).wait()
            pltpu.make_async_copy(dst_hbm.at[pl.ds(0, SUB)],
                                  dst_v.at[ib, t], isem).wait()
        pltpu.make_async_copy(wt_hbm.at[pl.ds(0, CH)], wt_v.at[ib],
                              isem).wait()

    def _rebase(ib):
        for t in range(NSUB):
            for u in range(SUB // LANES):
                sl = pl.ds(u * LANES, LANES)
                src_v[ib, t, sl] = src_v[ib, t, sl] * 2 + c

    def _fire_gathers(ib, rb):
        for t in range(NSUB):
            pltpu.async_copy(xs_hbm.at[src_v.at[ib, t]],
                             rows_v.at[rb, pl.ds(t * SUB, SUB)], gsem)

    def _drain_scatters():
        pltpu.make_async_copy(xs_hbm.at[pl.ds(0, CH)], rows_v.at[0],
                              ssem).wait()

    _load_idx(0, 0)
    _wait_idx(0)
    _rebase(0)
    _fire_gathers(0, 0)
    _load_idx(1, 1)

    for t in range(RPT // ZR):
        pltpu.make_async_copy(xs_hbm.at[pl.ds(0, ZR)], zero_v, ssem).wait()
    plsc.subcore_barrier()

    def chunk(k, carry):
        rb = lax.rem(k, NRB)
        nrb = 1 - rb
        ib = lax.rem(k, NIB)
        ib1 = lax.rem(k + 1, NIB)

        def egroup(g, ecarry):
            e0 = g * LANES
            wv = wt_v[ib, pl.ds(e0, LANES)]
            for l in range(LANES):
                w = wv[l]
                for j in range(DH // LANES):
                    rows_v[rb, e0 + l, pl.ds(j * LANES, LANES)] = (
                        rows_v[rb, e0 + l, pl.ds(j * LANES, LANES)] * w)
            return ecarry

        for t in range(NSUB):
            pltpu.make_async_copy(xs_hbm.at[pl.ds(0, SUB)],
                                  rows_v.at[rb, pl.ds(t * SUB, SUB)],
                                  gsem).wait()
            lax.fori_loop(t * (SUB // LANES), (t + 1) * (SUB // LANES),
                          egroup, 0)
            pltpu.async_copy(rows_v.at[rb, pl.ds(t * SUB, SUB)],
                             agg_sh.at[dst_v.at[ib, t]], ssem, add=True)

        @pl.when(k + 1 < NCHUNK)
        def _prep_next():
            _wait_idx(ib1)
            _rebase(ib1)

            @pl.when(k >= 1)
            def _drain_prev_scatters():
                _drain_scatters()

            _fire_gathers(ib1, nrb)

            @pl.when(k + 2 < NCHUNK)
            def _load_next_idx():
                _load_idx(k + 2, lax.rem(k + 2, NIB))

        return carry

    lax.fori_loop(0, NCHUNK, chunk, 0)

    _drain_scatters()
    _drain_scatters()
    plsc.subcore_barrier()

    pltpu.sync_copy(agg_sh.at[pl.ds(row0, RPT)],
                    out_hbm.at[pl.ds(c * N + row0, RPT)])

    @pl.when(s == NS - 1)
    def _write_tail():
        pltpu.sync_copy(agg_sh.at[pl.ds(NS * RPT, TAIL)],
                        out_hbm.at[pl.ds(c * N + NS * RPT, TAIL)])


BM = 1000


def _tc_body(p0_ref, p1_ref, w0_ref, w1_ref, b_ref, o_ref):
    o_ref[...] = (
        jnp.dot(p0_ref[...], w0_ref[...], preferred_element_type=jnp.float32)
        + jnp.dot(p1_ref[...], w1_ref[...], preferred_element_type=jnp.float32)
        + b_ref[...])


def _tc_matmul(partial, W, b):
    nb = N // BM
    return pl.pallas_call(
        _tc_body,
        grid=(nb,),
        in_specs=[
            pl.BlockSpec((BM, DH), lambda i: (i, 0)),
            pl.BlockSpec((BM, DH), lambda i: (i + nb, 0)),
            pl.BlockSpec((DH, D), lambda i: (0, 0)),
            pl.BlockSpec((DH, D), lambda i: (0, 0)),
            pl.BlockSpec((1, D), lambda i: (0, 0)),
        ],
        out_specs=pl.BlockSpec((BM, D), lambda i: (i, 0)),
        out_shape=jax.ShapeDtypeStruct((N, D), jnp.float32),
    )(partial, partial, W[:DH], W[DH:], b.reshape(1, D))


def kernel(x, edge_index, edge_weight, W, b):
    src = edge_index[1].astype(jnp.int32)
    dst = edge_index[0].astype(jnp.int32)
    xs = x.reshape(NC * N, DH)
    partial = _sc_agg(xs, src, dst, edge_weight)
    return _tc_matmul(partial, W, b)


# speedup vs baseline: 1.6022x; 1.0436x over previous
import functools

import jax
import jax.numpy as jnp
from jax import lax
from jax.experimental import pallas as pl
from jax.experimental.pallas import tpu as pltpu
from jax.experimental.pallas import tpu_sc as plsc

N = 10000
E = 320000
D = 128
DH = D // 2

NC = 2
NS = 16
EPT = E // NS
CH = 400
NCHUNK = EPT // CH
SUB = 80
NSUB = CH // SUB
NRB = 2
NIB = 3
RPT = 624
TAIL = N - NS * RPT
ZR = 104
LANES = 16

_mesh = plsc.VectorSubcoreMesh(core_axis_name="c", subcore_axis_name="s")


@functools.partial(
    pl.kernel,
    out_type=jax.ShapeDtypeStruct((NC * N, DH), jnp.float32),
    mesh=_mesh,
    scratch_types=[
        pltpu.VMEM((NIB, NSUB, SUB), jnp.int32),
        pltpu.VMEM((NIB, NSUB, SUB), jnp.int32),
        pltpu.VMEM((NIB, CH), jnp.float32),
        pltpu.VMEM((NRB, CH, DH), jnp.float32),
        pltpu.VMEM((ZR, DH), jnp.float32),
        pltpu.VMEM_SHARED((N, DH), jnp.float32),
        pltpu.SemaphoreType.DMA,
        pltpu.SemaphoreType.DMA,
        pltpu.SemaphoreType.DMA,
    ],
    compiler_params=pltpu.CompilerParams(use_tc_tiling_on_sc=False),
)
def _sc_agg(xs_hbm, ei_hbm, wt_hbm, out_hbm,
            src_v, dst_v, wt_v, rows_v, zero_v, agg_sh, isem, gsem, ssem):
    c = lax.axis_index("c")
    s = lax.axis_index("s")
    ebase = s * EPT

    zvec = jnp.zeros((LANES,), jnp.float32)

    def zrow(r, carry):
        for j in range(DH // LANES):
            zero_v[r, pl.ds(j * LANES, LANES)] = zvec
        return carry

    lax.fori_loop(0, ZR, zrow, 0)
    row0 = s * RPT
    for t in range(RPT // ZR):
        pltpu.async_copy(zero_v, agg_sh.at[pl.ds(row0 + t * ZR, ZR)], ssem)

    @pl.when(s == NS - 1)
    def _zero_tail():
        pltpu.sync_copy(zero_v.at[pl.ds(0, TAIL)],
                        agg_sh.at[pl.ds(NS * RPT, TAIL)])

    def _load_idx(k, ib):
        base = ebase + k * CH
        for t in range(NSUB):
            pltpu.async_copy(ei_hbm.at[1, pl.ds(base + t * SUB, SUB)],
                             src_v.at[ib, t], isem)
            pltpu.async_copy(ei_hbm.at[0, pl.ds(base + t * SUB, SUB)],
                             dst_v.at[ib, t], isem)
        pltpu.async_copy(wt_hbm.at[pl.ds(base, CH)], wt_v.at[ib], isem)

    def _wait_idx(ib):
        for t in range(NSUB):
            pltpu.make_async_copy(ei_hbm.at[1, pl.ds(0, SUB)],
                                  src_v.at[ib, t], isem).wait()
            pltpu.make_async_copy(ei_hbm.at[0, pl.ds(0, SUB)],
                                  dst_v.at[ib, t], isem).wait()
        pltpu.make_async_copy(wt_hbm.at[pl.ds(0, CH)], wt_v.at[ib],
                              isem).wait()

    def _rebase(ib):
        for t in range(NSUB):
            for u in range(SUB // LANES):
                sl = pl.ds(u * LANES, LANES)
                src_v[ib, t, sl] = src_v[ib, t, sl] * 2 + c

    def _fire_gathers(ib, rb):
        for t in range(NSUB):
            pltpu.async_copy(xs_hbm.at[src_v.at[ib, t]],
                             rows_v.at[rb, pl.ds(t * SUB, SUB)], gsem)

    def _drain_scatters():
        pltpu.make_async_copy(xs_hbm.at[pl.ds(0, CH)], rows_v.at[0],
                              ssem).wait()

    _load_idx(0, 0)
    _wait_idx(0)
    _rebase(0)
    _fire_gathers(0, 0)
    _load_idx(1, 1)

    for t in range(RPT // ZR):
        pltpu.make_async_copy(xs_hbm.at[pl.ds(0, ZR)], zero_v, ssem).wait()
    plsc.subcore_barrier()

    def chunk(k, carry):
        rb = lax.rem(k, NRB)
        nrb = 1 - rb
        ib = lax.rem(k, NIB)
        ib1 = lax.rem(k + 1, NIB)

        def egroup(g, ecarry):
            e0 = g * LANES
            wv = wt_v[ib, pl.ds(e0, LANES)]
            for l in range(LANES):
                w = wv[l]
                for j in range(DH // LANES):
                    rows_v[rb, e0 + l, pl.ds(j * LANES, LANES)] = (
                        rows_v[rb, e0 + l, pl.ds(j * LANES, LANES)] * w)
            return ecarry

        for t in range(NSUB):
            pltpu.make_async_copy(xs_hbm.at[pl.ds(0, SUB)],
                                  rows_v.at[rb, pl.ds(t * SUB, SUB)],
                                  gsem).wait()
            lax.fori_loop(t * (SUB // LANES), (t + 1) * (SUB // LANES),
                          egroup, 0)
            pltpu.async_copy(rows_v.at[rb, pl.ds(t * SUB, SUB)],
                             agg_sh.at[dst_v.at[ib, t]], ssem, add=True)

        @pl.when(k + 1 < NCHUNK)
        def _prep_next():
            _wait_idx(ib1)
            _rebase(ib1)

            @pl.when(k >= 1)
            def _drain_prev_scatters():
                _drain_scatters()

            _fire_gathers(ib1, nrb)

            @pl.when(k + 2 < NCHUNK)
            def _load_next_idx():
                _load_idx(k + 2, lax.rem(k + 2, NIB))

        return carry

    lax.fori_loop(0, NCHUNK, chunk, 0)

    _drain_scatters()
    _drain_scatters()
    plsc.subcore_barrier()

    pltpu.sync_copy(agg_sh.at[pl.ds(row0, RPT)],
                    out_hbm.at[pl.ds(c * N + row0, RPT)])

    @pl.when(s == NS - 1)
    def _write_tail():
        pltpu.sync_copy(agg_sh.at[pl.ds(NS * RPT, TAIL)],
                        out_hbm.at[pl.ds(c * N + NS * RPT, TAIL)])


BM = 1000


def _tc_body(p0_ref, p1_ref, w0_ref, w1_ref, b_ref, o_ref):
    o_ref[...] = (
        jnp.dot(p0_ref[...], w0_ref[...], preferred_element_type=jnp.float32)
        + jnp.dot(p1_ref[...], w1_ref[...], preferred_element_type=jnp.float32)
        + b_ref[...])


def _tc_matmul(partial, W, b):
    nb = N // BM
    return pl.pallas_call(
        _tc_body,
        grid=(nb,),
        in_specs=[
            pl.BlockSpec((BM, DH), lambda i: (i, 0)),
            pl.BlockSpec((BM, DH), lambda i: (i + nb, 0)),
            pl.BlockSpec((DH, D), lambda i: (0, 0)),
            pl.BlockSpec((DH, D), lambda i: (0, 0)),
            pl.BlockSpec((1, D), lambda i: (0, 0)),
        ],
        out_specs=pl.BlockSpec((BM, D), lambda i: (i, 0)),
        out_shape=jax.ShapeDtypeStruct((N, D), jnp.float32),
    )(partial, partial, W[:DH], W[DH:], b.reshape(1, D))


def kernel(x, edge_index, edge_weight, W, b):
    ei = jnp.asarray(edge_index, dtype=jnp.int32)
    xs = x.reshape(NC * N, DH)
    partial = _sc_agg(xs, ei, edge_weight)
    return _tc_matmul(partial, W, b)


# speedup vs baseline: 1.9470x; 1.2152x over previous
import functools

import jax
import jax.numpy as jnp
from jax import lax
from jax.experimental import pallas as pl
from jax.experimental.pallas import tpu as pltpu
from jax.experimental.pallas import tpu_sc as plsc

N = 10000
E = 320000
D = 128
DH = D // 2

NC = 2
NS = 16
EPT = E // NS
CH = 400
NCHUNK = EPT // CH
SUB = 80
NSUB = CH // SUB
NRB = 3
NIB = 4
RPT = 624
TAIL = N - NS * RPT
ZR = 104
LANES = 16

_mesh = plsc.VectorSubcoreMesh(core_axis_name="c", subcore_axis_name="s")


@functools.partial(
    pl.kernel,
    out_type=jax.ShapeDtypeStruct((NC * N, DH), jnp.float32),
    mesh=_mesh,
    scratch_types=[
        pltpu.VMEM((NIB, NSUB, SUB), jnp.int32),
        pltpu.VMEM((NIB, NSUB, SUB), jnp.int32),
        pltpu.VMEM((NIB, CH), jnp.float32),
        pltpu.VMEM((NRB, CH, DH), jnp.float32),
        pltpu.VMEM((ZR, DH), jnp.float32),
        pltpu.VMEM_SHARED((N, DH), jnp.float32),
        pltpu.SemaphoreType.DMA,
        pltpu.SemaphoreType.DMA,
        pltpu.SemaphoreType.DMA,
    ],
    compiler_params=pltpu.CompilerParams(use_tc_tiling_on_sc=False),
)
def _sc_agg(xs_hbm, ei_hbm, wt_hbm, out_hbm,
            src_v, dst_v, wt_v, rows_v, zero_v, agg_sh, isem, gsem, ssem):
    c = lax.axis_index("c")
    s = lax.axis_index("s")
    ebase = s * EPT

    zvec = jnp.zeros((LANES,), jnp.float32)

    def zrow(r, carry):
        for j in range(DH // LANES):
            zero_v[r, pl.ds(j * LANES, LANES)] = zvec
        return carry

    lax.fori_loop(0, ZR, zrow, 0)
    row0 = s * RPT
    for t in range(RPT // ZR):
        pltpu.async_copy(zero_v, agg_sh.at[pl.ds(row0 + t * ZR, ZR)], ssem)

    @pl.when(s == NS - 1)
    def _zero_tail():
        pltpu.sync_copy(zero_v.at[pl.ds(0, TAIL)],
                        agg_sh.at[pl.ds(NS * RPT, TAIL)])

    def _load_idx(k, ib):
        base = ebase + k * CH
        for t in range(NSUB):
            pltpu.async_copy(ei_hbm.at[1, pl.ds(base + t * SUB, SUB)],
                             src_v.at[ib, t], isem)
            pltpu.async_copy(ei_hbm.at[0, pl.ds(base + t * SUB, SUB)],
                             dst_v.at[ib, t], isem)
        pltpu.async_copy(wt_hbm.at[pl.ds(base, CH)], wt_v.at[ib], isem)

    def _wait_idx(ib):
        for t in range(NSUB):
            pltpu.make_async_copy(ei_hbm.at[1, pl.ds(0, SUB)],
                                  src_v.at[ib, t], isem).wait()
            pltpu.make_async_copy(ei_hbm.at[0, pl.ds(0, SUB)],
                                  dst_v.at[ib, t], isem).wait()
        pltpu.make_async_copy(wt_hbm.at[pl.ds(0, CH)], wt_v.at[ib],
                              isem).wait()

    def _rebase(ib):
        for t in range(NSUB):
            for u in range(SUB // LANES):
                sl = pl.ds(u * LANES, LANES)
                src_v[ib, t, sl] = src_v[ib, t, sl] * 2 + c

    def _fire_gathers(ib, rb):
        for t in range(NSUB):
            pltpu.async_copy(xs_hbm.at[src_v.at[ib, t]],
                             rows_v.at[rb, pl.ds(t * SUB, SUB)], gsem)

    def _drain_scatters():
        pltpu.make_async_copy(xs_hbm.at[pl.ds(0, CH)], rows_v.at[0],
                              ssem).wait()

    _load_idx(0, 0)
    _load_idx(1, 1)
    _load_idx(2, 2)
    _wait_idx(0)
    _rebase(0)
    _fire_gathers(0, 0)
    _wait_idx(1)
    _rebase(1)
    _fire_gathers(1, 1)

    for t in range(RPT // ZR):
        pltpu.make_async_copy(xs_hbm.at[pl.ds(0, ZR)], zero_v, ssem).wait()
    plsc.subcore_barrier()

    def chunk(k, rb):
        rb2 = jnp.where(rb >= 1, rb - 1, rb + 2)
        ib = jnp.bitwise_and(k, NIB - 1)
        ib2 = jnp.bitwise_and(k + 2, NIB - 1)
        ib3 = jnp.bitwise_and(k + 3, NIB - 1)

        def egroup(g, ecarry):
            e0 = g * LANES
            wv = wt_v[ib, pl.ds(e0, LANES)]
            for l in range(LANES):
                w = wv[l]
                for j in range(DH // LANES):
                    rows_v[rb, e0 + l, pl.ds(j * LANES, LANES)] = (
                        rows_v[rb, e0 + l, pl.ds(j * LANES, LANES)] * w)
            return ecarry

        for t in range(NSUB):
            pltpu.make_async_copy(xs_hbm.at[pl.ds(0, SUB)],
                                  rows_v.at[rb, pl.ds(t * SUB, SUB)],
                                  gsem).wait()
            lax.fori_loop(t * (SUB // LANES), (t + 1) * (SUB // LANES),
                          egroup, 0)
            pltpu.async_copy(rows_v.at[rb, pl.ds(t * SUB, SUB)],
                             agg_sh.at[dst_v.at[ib, t]], ssem, add=True)

        @pl.when(k + 2 < NCHUNK)
        def _prep_next():
            _wait_idx(ib2)
            _rebase(ib2)

            @pl.when(k >= 1)
            def _drain_prev_scatters():
                _drain_scatters()

            _fire_gathers(ib2, rb2)

            @pl.when(k + 3 < NCHUNK)
            def _load_next_idx():
                _load_idx(k + 3, ib3)

        return jnp.where(rb >= NRB - 1, 0, rb + 1)

    lax.fori_loop(0, NCHUNK, chunk, jnp.int32(0))

    _drain_scatters()
    _drain_scatters()
    _drain_scatters()
    plsc.subcore_barrier()

    pltpu.sync_copy(agg_sh.at[pl.ds(row0, RPT)],
                    out_hbm.at[pl.ds(c * N + row0, RPT)])

    @pl.when(s == NS - 1)
    def _write_tail():
        pltpu.sync_copy(agg_sh.at[pl.ds(NS * RPT, TAIL)],
                        out_hbm.at[pl.ds(c * N + NS * RPT, TAIL)])


BM = 1000


def _tc_body(p0_ref, p1_ref, w0_ref, w1_ref, b_ref, o_ref):
    o_ref[...] = (
        jnp.dot(p0_ref[...], w0_ref[...], preferred_element_type=jnp.float32)
        + jnp.dot(p1_ref[...], w1_ref[...], preferred_element_type=jnp.float32)
        + b_ref[...])


def _tc_matmul(partial, W, b):
    nb = N // BM
    return pl.pallas_call(
        _tc_body,
        grid=(nb,),
        in_specs=[
            pl.BlockSpec((BM, DH), lambda i: (i, 0)),
            pl.BlockSpec((BM, DH), lambda i: (i + nb, 0)),
            pl.BlockSpec((DH, D), lambda i: (0, 0)),
            pl.BlockSpec((DH, D), lambda i: (0, 0)),
            pl.BlockSpec((1, D), lambda i: (0, 0)),
        ],
        out_specs=pl.BlockSpec((BM, D), lambda i: (i, 0)),
        out_shape=jax.ShapeDtypeStruct((N, D), jnp.float32),
    )(partial, partial, W[:DH], W[DH:], b.reshape(1, D))


def kernel(x, edge_index, edge_weight, W, b):
    ei = jnp.asarray(edge_index, dtype=jnp.int32)
    xs = x.reshape(NC * N, DH)
    partial = _sc_agg(xs, ei, edge_weight)
    return _tc_matmul(partial, W, b)


# speedup vs baseline: 2.0646x; 1.0604x over previous
import functools

import jax
import jax.numpy as jnp
from jax import lax
from jax.experimental import pallas as pl
from jax.experimental.pallas import tpu as pltpu
from jax.experimental.pallas import tpu_sc as plsc

N = 10000
E = 320000
D = 128
DH = D // 2

NC = 2
NS = 16
EPT = E // NS
CH = 400
NCHUNK = EPT // CH
SUB = 80
NSUB = CH // SUB
NRB = 3
NIB = 4
RPT = 624
TAIL = N - NS * RPT
ZR = 104
LANES = 16

_mesh = plsc.VectorSubcoreMesh(core_axis_name="c", subcore_axis_name="s")


@functools.partial(
    pl.kernel,
    out_type=jax.ShapeDtypeStruct((NC * N, DH), jnp.float32),
    mesh=_mesh,
    scratch_types=[
        pltpu.VMEM((NIB, NSUB, SUB), jnp.int32),
        pltpu.VMEM((NIB, NSUB, SUB), jnp.int32),
        pltpu.VMEM((NIB, CH), jnp.float32),
        pltpu.VMEM((NRB, CH, DH), jnp.float32),
        pltpu.VMEM((ZR, DH), jnp.float32),
        pltpu.VMEM_SHARED((N, DH), jnp.float32),
        pltpu.SemaphoreType.DMA,
        pltpu.SemaphoreType.DMA,
        pltpu.SemaphoreType.DMA,
    ],
    compiler_params=pltpu.CompilerParams(use_tc_tiling_on_sc=False),
)
def _sc_agg(xs_hbm, ei_hbm, wt_hbm, out_hbm,
            src_v, dst_v, wt_v, rows_v, zero_v, agg_sh, isem, gsem, ssem):
    c = lax.axis_index("c")
    s = lax.axis_index("s")
    ebase = s * EPT

    zvec = jnp.zeros((LANES,), jnp.float32)

    def zrow(r, carry):
        for j in range(DH // LANES):
            zero_v[r, pl.ds(j * LANES, LANES)] = zvec
        return carry

    lax.fori_loop(0, ZR, zrow, 0)
    row0 = s * RPT
    for t in range(RPT // ZR):
        pltpu.async_copy(zero_v, agg_sh.at[pl.ds(row0 + t * ZR, ZR)], ssem)

    @pl.when(s == NS - 1)
    def _zero_tail():
        pltpu.sync_copy(zero_v.at[pl.ds(0, TAIL)],
                        agg_sh.at[pl.ds(NS * RPT, TAIL)])

    def _load_idx(k, ib):
        base = ebase + k * CH
        for t in range(NSUB):
            pltpu.async_copy(ei_hbm.at[1, pl.ds(base + t * SUB, SUB)],
                             src_v.at[ib, t], isem)
            pltpu.async_copy(ei_hbm.at[0, pl.ds(base + t * SUB, SUB)],
                             dst_v.at[ib, t], isem)
        pltpu.async_copy(wt_hbm.at[pl.ds(base, CH)], wt_v.at[ib], isem)

    def _wait_idx(ib):
        for t in range(NSUB):
            pltpu.make_async_copy(ei_hbm.at[1, pl.ds(0, SUB)],
                                  src_v.at[ib, t], isem).wait()
            pltpu.make_async_copy(ei_hbm.at[0, pl.ds(0, SUB)],
                                  dst_v.at[ib, t], isem).wait()
        pltpu.make_async_copy(wt_hbm.at[pl.ds(0, CH)], wt_v.at[ib],
                              isem).wait()

    def _rebase(ib):
        for t in range(NSUB):
            for u in range(SUB // LANES):
                sl = pl.ds(u * LANES, LANES)
                src_v[ib, t, sl] = src_v[ib, t, sl] * 2 + c

    def _fire_gathers(ib, rb):
        for t in range(NSUB):
            pltpu.async_copy(xs_hbm.at[src_v.at[ib, t]],
                             rows_v.at[rb, pl.ds(t * SUB, SUB)], gsem)

    def _drain_scatters():
        pltpu.make_async_copy(xs_hbm.at[pl.ds(0, CH)], rows_v.at[0],
                              ssem).wait()

    _load_idx(0, 0)
    _load_idx(1, 1)
    _load_idx(2, 2)
    _wait_idx(0)
    _rebase(0)
    _fire_gathers(0, 0)
    _wait_idx(1)
    _rebase(1)
    _fire_gathers(1, 1)

    for t in range(RPT // ZR):
        pltpu.make_async_copy(xs_hbm.at[pl.ds(0, ZR)], zero_v, ssem).wait()
    plsc.subcore_barrier()

    def chunk(k, rb):
        rb2 = jnp.where(rb >= 1, rb - 1, rb + 2)
        ib = jnp.bitwise_and(k, NIB - 1)
        ib2 = jnp.bitwise_and(k + 2, NIB - 1)
        ib3 = jnp.bitwise_and(k + 3, NIB - 1)

        def egroup(g, ecarry):
            e0 = g * LANES
            wv = wt_v[ib, pl.ds(e0, LANES)]
            for l in range(LANES):
                w = wv[l]
                for j in range(DH // LANES):
                    rows_v[rb, e0 + l, pl.ds(j * LANES, LANES)] = (
                        rows_v[rb, e0 + l, pl.ds(j * LANES, LANES)] * w)
            return ecarry

        for t in range(NSUB):
            pltpu.make_async_copy(xs_hbm.at[pl.ds(0, SUB)],
                                  rows_v.at[rb, pl.ds(t * SUB, SUB)],
                                  gsem).wait()
            lax.fori_loop(t * (SUB // LANES), (t + 1) * (SUB // LANES),
                          egroup, 0, unroll=SUB // LANES)
            pltpu.async_copy(rows_v.at[rb, pl.ds(t * SUB, SUB)],
                             agg_sh.at[dst_v.at[ib, t]], ssem, add=True)

        @pl.when(k + 2 < NCHUNK)
        def _prep_next():
            _wait_idx(ib2)
            _rebase(ib2)

            @pl.when(k >= 1)
            def _drain_prev_scatters():
                _drain_scatters()

            _fire_gathers(ib2, rb2)

            @pl.when(k + 3 < NCHUNK)
            def _load_next_idx():
                _load_idx(k + 3, ib3)

        return jnp.where(rb >= NRB - 1, 0, rb + 1)

    lax.fori_loop(0, NCHUNK, chunk, jnp.int32(0))

    _drain_scatters()
    _drain_scatters()
    _drain_scatters()
    plsc.subcore_barrier()

    pltpu.sync_copy(agg_sh.at[pl.ds(row0, RPT)],
                    out_hbm.at[pl.ds(c * N + row0, RPT)])

    @pl.when(s == NS - 1)
    def _write_tail():
        pltpu.sync_copy(agg_sh.at[pl.ds(NS * RPT, TAIL)],
                        out_hbm.at[pl.ds(c * N + NS * RPT, TAIL)])


BM = 1000


def _tc_body(p0_ref, p1_ref, w0_ref, w1_ref, b_ref, o_ref):
    o_ref[...] = (
        jnp.dot(p0_ref[...], w0_ref[...], preferred_element_type=jnp.float32)
        + jnp.dot(p1_ref[...], w1_ref[...], preferred_element_type=jnp.float32)
        + b_ref[...])


def _tc_matmul(partial, W, b):
    nb = N // BM
    return pl.pallas_call(
        _tc_body,
        grid=(nb,),
        in_specs=[
            pl.BlockSpec((BM, DH), lambda i: (i, 0)),
            pl.BlockSpec((BM, DH), lambda i: (i + nb, 0)),
            pl.BlockSpec((DH, D), lambda i: (0, 0)),
            pl.BlockSpec((DH, D), lambda i: (0, 0)),
            pl.BlockSpec((1, D), lambda i: (0, 0)),
        ],
        out_specs=pl.BlockSpec((BM, D), lambda i: (i, 0)),
        out_shape=jax.ShapeDtypeStruct((N, D), jnp.float32),
    )(partial, partial, W[:DH], W[DH:], b.reshape(1, D))


def kernel(x, edge_index, edge_weight, W, b):
    ei = jnp.asarray(edge_index, dtype=jnp.int32)
    xs = x.reshape(NC * N, DH)
    partial = _sc_agg(xs, ei, edge_weight)
    return _tc_matmul(partial, W, b)
